# Initial kernel scaffold; baseline (speedup 1.0000x reference)
import jax
import jax.numpy as jnp
from jax.experimental import pallas as pl


def kernel(x, edge_index, W1, b1, W2, b2):
    raise NotImplementedError("write your pallas kernel here")


# speedup vs baseline: 9.1704x; 9.1704x over previous
import functools

import jax
import jax.numpy as jnp
from jax import lax
from jax.experimental import pallas as pl
from jax.experimental.pallas import tpu as pltpu
from jax.experimental.pallas import tpu_sc as plsc

N = 10000
E = 320000
C = 128
NC, NS = 2, 16
E_PAD = 79 * NC * NS * C
EPT_CHUNKS = E_PAD // (NS * C)
EPT = EPT_CHUNKS * C
EPT2_CHUNKS = E_PAD // (NC * NS * C)
EPT2 = EPT2_CHUNKS * C
NPAD = N + 8
SLAB = 624
TAIL = N - NS * SLAB
R = 400
NB = N // R


def _mesh():
    return plsc.VectorSubcoreMesh(
        core_axis_name="c", subcore_axis_name="s", num_cores=NC, num_subcores=NS
    )


def _sc_deg(dst_pad, zeros128, ones128):
    D = 128

    @functools.partial(
        pl.kernel,
        out_type=jax.ShapeDtypeStruct((2 * N, D), jnp.float32),
        mesh=_mesh(),
        scratch_types=[
            pltpu.VMEM((C,), jnp.int32),
            pltpu.VMEM((C, D), jnp.float32),
            pltpu.VMEM_SHARED((NPAD, D), jnp.float32),
        ],
    )
    def k(dst_hbm, z_hbm, one_hbm, out_hbm, idx_v, ones_v, acc):
        c = lax.axis_index("c")
        s = lax.axis_index("s")
        pltpu.sync_copy(z_hbm.at[pl.ds(s * SLAB, SLAB)], acc.at[pl.ds(s * SLAB, SLAB)])

        @pl.when(s == 0)
        def _():
            pltpu.sync_copy(
                z_hbm.at[pl.ds(NS * SLAB, TAIL)], acc.at[pl.ds(NS * SLAB, TAIL)]
            )

        pltpu.sync_copy(one_hbm, ones_v)
        plsc.subcore_barrier()

        def body(i, carry):
            base = (c * NS + s) * EPT2 + i * C
            pltpu.sync_copy(dst_hbm.at[pl.ds(base, C)], idx_v)
            pltpu.sync_copy(ones_v, acc.at[idx_v], add=True)
            return carry

        lax.fori_loop(0, EPT2_CHUNKS, body, 0)
        plsc.subcore_barrier()
        pltpu.sync_copy(
            acc.at[pl.ds(s * SLAB, SLAB)], out_hbm.at[pl.ds(c * N + s * SLAB, SLAB)]
        )

        @pl.when(s == 0)
        def _():
            pltpu.sync_copy(
                acc.at[pl.ds(NS * SLAB, TAIL)],
                out_hbm.at[pl.ds(c * N + NS * SLAB, TAIL)],
            )

    return k(dst_pad, zeros128, ones128)


def _sc_agg(y_cat, src_flat, dst_pad):
    D = 128

    @functools.partial(
        pl.kernel,
        out_type=jax.ShapeDtypeStruct((2 * N, D), jnp.float32),
        mesh=_mesh(),
        scratch_types=[
            pltpu.VMEM((C,), jnp.int32),
            pltpu.VMEM((C,), jnp.int32),
            pltpu.VMEM((C, D), jnp.float32),
            pltpu.VMEM_SHARED((NPAD, D), jnp.float32),
            pltpu.SemaphoreType.DMA,
        ],
    )
    def k(y_hbm, src_hbm, dst_hbm, out_hbm, sidx_v, didx_v, rows_v, acc, sem):
        c = lax.axis_index("c")
        s = lax.axis_index("s")
        pltpu.sync_copy(
            y_hbm.at[pl.ds(c * N + s * SLAB, SLAB)], acc.at[pl.ds(s * SLAB, SLAB)]
        )

        @pl.when(s == 0)
        def _():
            pltpu.sync_copy(
                y_hbm.at[pl.ds(c * N + NS * SLAB, TAIL)],
                acc.at[pl.ds(NS * SLAB, TAIL)],
            )

        plsc.subcore_barrier()

        def body(i, carry):
            e = s * EPT + i * C
            pltpu.sync_copy(src_hbm.at[pl.ds(c * E_PAD + e, C)], sidx_v)
            pltpu.sync_copy(dst_hbm.at[pl.ds(e, C)], didx_v)
            pltpu.async_copy(y_hbm.at[sidx_v], rows_v, sem).wait()
            pltpu.sync_copy(rows_v, acc.at[didx_v], add=True)
            return carry

        lax.fori_loop(0, EPT_CHUNKS, body, 0)
        plsc.subcore_barrier()
        pltpu.sync_copy(
            acc.at[pl.ds(s * SLAB, SLAB)], out_hbm.at[pl.ds(c * N + s * SLAB, SLAB)]
        )

        @pl.when(s == 0)
        def _():
            pltpu.sync_copy(
                acc.at[pl.ds(NS * SLAB, TAIL)],
                out_hbm.at[pl.ds(c * N + NS * SLAB, TAIL)],
            )

    return k(y_cat, src_flat, dst_pad)


def _sc_agg_edges(y, src_pad, dst_pad, zeros128):
    D = 128

    @functools.partial(
        pl.kernel,
        out_type=jax.ShapeDtypeStruct((2 * N, D), jnp.float32),
        mesh=_mesh(),
        scratch_types=[
            pltpu.VMEM((C,), jnp.int32),
            pltpu.VMEM((C,), jnp.int32),
            pltpu.VMEM((C, D), jnp.float32),
            pltpu.VMEM_SHARED((NPAD, D), jnp.float32),
            pltpu.SemaphoreType.DMA,
        ],
    )
    def k(y_hbm, z_hbm, src_hbm, dst_hbm, out_hbm, sidx_v, didx_v, rows_v, acc, sem):
        c = lax.axis_index("c")
        s = lax.axis_index("s")
        init = pl.ds(s * SLAB, SLAB)

        @pl.when(c == 0)
        def _():
            pltpu.sync_copy(y_hbm.at[init], acc.at[init])

            @pl.when(s == 0)
            def _():
                pltpu.sync_copy(
                    y_hbm.at[pl.ds(NS * SLAB, TAIL)], acc.at[pl.ds(NS * SLAB, TAIL)]
                )

        @pl.when(c == 1)
        def _():
            pltpu.sync_copy(z_hbm.at[init], acc.at[init])

            @pl.when(s == 0)
            def _():
                pltpu.sync_copy(
                    z_hbm.at[pl.ds(NS * SLAB, TAIL)], acc.at[pl.ds(NS * SLAB, TAIL)]
                )

        plsc.subcore_barrier()

        def body(i, carry):
            e = (c * NS + s) * EPT2 + i * C
            pltpu.sync_copy(src_hbm.at[pl.ds(e, C)], sidx_v)
            pltpu.sync_copy(dst_hbm.at[pl.ds(e, C)], didx_v)
            pltpu.async_copy(y_hbm.at[sidx_v], rows_v, sem).wait()
            pltpu.sync_copy(rows_v, acc.at[didx_v], add=True)
            return carry

        lax.fori_loop(0, EPT2_CHUNKS, body, 0)
        plsc.subcore_barrier()
        pltpu.sync_copy(
            acc.at[pl.ds(s * SLAB, SLAB)], out_hbm.at[pl.ds(c * N + s * SLAB, SLAB)]
        )

        @pl.when(s == 0)
        def _():
            pltpu.sync_copy(
                acc.at[pl.ds(NS * SLAB, TAIL)],
                out_hbm.at[pl.ds(c * N + NS * SLAB, TAIL)],
            )

    return k(y, zeros128, src_pad, dst_pad)


def _tc_scale_mm(x, w, degp):
    K = x.shape[1]
    F = w.shape[1] // 2

    def body(x_ref, w_ref, dlo_ref, dhi_ref, o_ref):
        dis = lax.rsqrt(dlo_ref[:, 0:1] + dhi_ref[:, 0:1] + 1.0)
        o_ref[...] = dis * jnp.dot(
            x_ref[...], w_ref[...], preferred_element_type=jnp.float32
        )

    return pl.pallas_call(
        body,
        grid=(NB, 2),
        in_specs=[
            pl.BlockSpec((R, K), lambda i, c: (i, 0)),
            pl.BlockSpec((K, F), lambda i, c: (0, c)),
            pl.BlockSpec((R, 128), lambda i, c: (i, 0)),
            pl.BlockSpec((R, 128), lambda i, c: (NB + i, 0)),
        ],
        out_specs=pl.BlockSpec((R, F), lambda i, c: (c * NB + i, 0)),
        out_shape=jax.ShapeDtypeStruct((2 * N, F), jnp.float32),
    )(x, w, degp, degp)


def _tc_mid(agg, degp, b1, w2):
    def body(lo_ref, hi_ref, dlo_ref, dhi_ref, b_ref, w_ref, o_ref):
        dis = lax.rsqrt(dlo_ref[:, 0:1] + dhi_ref[:, 0:1] + 1.0)
        h = jnp.concatenate([lo_ref[...], hi_ref[...]], axis=1)
        h = jnp.maximum(dis * h + b_ref[...], 0.0)
        o_ref[...] = dis * jnp.dot(h, w_ref[...], preferred_element_type=jnp.float32)

    return pl.pallas_call(
        body,
        grid=(NB,),
        in_specs=[
            pl.BlockSpec((R, 128), lambda i: (i, 0)),
            pl.BlockSpec((R, 128), lambda i: (NB + i, 0)),
            pl.BlockSpec((R, 128), lambda i: (i, 0)),
            pl.BlockSpec((R, 128), lambda i: (NB + i, 0)),
            pl.BlockSpec((1, 256), lambda i: (0, 0)),
            pl.BlockSpec((256, 128), lambda i: (0, 0)),
        ],
        out_specs=pl.BlockSpec((R, 128), lambda i: (i, 0)),
        out_shape=jax.ShapeDtypeStruct((N, 128), jnp.float32),
    )(agg, agg, degp, degp, b1, w2)


def _tc_final(agg, degp, b2):
    def body(lo_ref, hi_ref, dlo_ref, dhi_ref, b_ref, o_ref):
        dis = lax.rsqrt(dlo_ref[:, 0:1] + dhi_ref[:, 0:1] + 1.0)
        h = lo_ref[...] + hi_ref[...]
        o_ref[...] = jnp.maximum(dis * h + b_ref[...], 0.0)

    return pl.pallas_call(
        body,
        grid=(NB,),
        in_specs=[
            pl.BlockSpec((R, 128), lambda i: (i, 0)),
            pl.BlockSpec((R, 128), lambda i: (NB + i, 0)),
            pl.BlockSpec((R, 128), lambda i: (i, 0)),
            pl.BlockSpec((R, 128), lambda i: (NB + i, 0)),
            pl.BlockSpec((1, 128), lambda i: (0, 0)),
        ],
        out_specs=pl.BlockSpec((R, 128), lambda i: (i, 0)),
        out_shape=jax.ShapeDtypeStruct((N, 128), jnp.float32),
    )(agg, agg, degp, degp, b2)


def kernel(x, edge_index, W1, b1, W2, b2):
    src = edge_index[0].astype(jnp.int32)
    dst = edge_index[1].astype(jnp.int32)
    pad = E_PAD - E
    src_p = jnp.concatenate([src, jnp.zeros((pad,), jnp.int32)])
    dst_p = jnp.concatenate([dst, jnp.full((pad,), N, jnp.int32)])
    src_flat = jnp.concatenate([src_p, src_p + N])
    zeros128 = jnp.zeros((N, 128), jnp.float32)
    ones128 = jnp.ones((C, 128), jnp.float32)

    degp = _sc_deg(dst_p, zeros128, ones128)
    y1 = _tc_scale_mm(x, W1, degp)
    a1 = _sc_agg(y1, src_flat, dst_p)
    y2 = _tc_mid(a1, degp, b1.reshape(1, -1), W2)
    a2 = _sc_agg_edges(y2, src_p, dst_p, zeros128)
    return _tc_final(a2, degp, b2.reshape(1, -1))


# speedup vs baseline: 9.2970x; 1.0138x over previous
import functools

import jax
import jax.numpy as jnp
from jax import lax
from jax.experimental import pallas as pl
from jax.experimental.pallas import tpu as pltpu
from jax.experimental.pallas import tpu_sc as plsc

N = 10000
E = 320000
C = 128
NC, NS = 2, 16
E_PAD = 80 * NC * NS * C
EPT = E_PAD // NS
EPT_CHUNKS = EPT // C
EPT2 = E_PAD // (NC * NS)
EPT2_CHUNKS = EPT2 // C
NPAD = N + 8
SLAB = 624
TAIL = N - NS * SLAB
R = 400
NB = N // R


def _mesh():
    return plsc.VectorSubcoreMesh(
        core_axis_name="c", subcore_axis_name="s", num_cores=NC, num_subcores=NS
    )


GRP = 16


def _agg_scratch(D):
    return (
        [pltpu.VMEM((GRP, C), jnp.int32)]
        + [pltpu.VMEM((GRP, C), jnp.int32)]
        + [pltpu.VMEM((C, D), jnp.float32) for _ in range(2)]
        + [pltpu.VMEM_SHARED((NPAD, D), jnp.float32)]
        + [pltpu.SemaphoreType.DMA for _ in range(2)]
    )


def _agg_pipeline(y_hbm, src2_hbm, dst2_hbm, acc, sidx2, didx2, rows, semg,
                  src_row, dst_row, nch):
    def body(g, carry):
        pltpu.sync_copy(src2_hbm.at[pl.ds(src_row + g * GRP, GRP)], sidx2)
        pltpu.sync_copy(dst2_hbm.at[pl.ds(dst_row + g * GRP, GRP)], didx2)
        descs = [
            pltpu.async_copy(y_hbm.at[sidx2.at[0]], rows[0], semg[0]),
            pltpu.async_copy(y_hbm.at[sidx2.at[1]], rows[1], semg[1]),
        ]
        for j in range(GRP):
            b = j % 2
            descs[b].wait()
            pltpu.sync_copy(rows[b], acc.at[didx2.at[j]], add=True)
            if j + 2 < GRP:
                descs[b] = pltpu.async_copy(
                    y_hbm.at[sidx2.at[j + 2]], rows[b], semg[b]
                )
        return carry

    lax.fori_loop(0, nch // GRP, body, 0)


def _slab_init(src_hbm, acc, s, src_off):
    pltpu.sync_copy(
        src_hbm.at[pl.ds(src_off + s * SLAB, SLAB)], acc.at[pl.ds(s * SLAB, SLAB)]
    )

    @pl.when(s == 0)
    def _():
        pltpu.sync_copy(
            src_hbm.at[pl.ds(src_off + NS * SLAB, TAIL)],
            acc.at[pl.ds(NS * SLAB, TAIL)],
        )


def _slab_out(acc, out_hbm, s, dst_off):
    pltpu.sync_copy(
        acc.at[pl.ds(s * SLAB, SLAB)], out_hbm.at[pl.ds(dst_off + s * SLAB, SLAB)]
    )

    @pl.when(s == 0)
    def _():
        pltpu.sync_copy(
            acc.at[pl.ds(NS * SLAB, TAIL)],
            out_hbm.at[pl.ds(dst_off + NS * SLAB, TAIL)],
        )


def _sc_deg(dst_pad, zeros128, ones128):
    D = 128

    @functools.partial(
        pl.kernel,
        out_type=jax.ShapeDtypeStruct((2 * N, D), jnp.float32),
        mesh=_mesh(),
        scratch_types=[
            pltpu.VMEM((GRP, C), jnp.int32),
            pltpu.VMEM((C, D), jnp.float32),
            pltpu.VMEM_SHARED((NPAD, D), jnp.float32),
        ],
    )
    def k(dst2_hbm, z_hbm, one_hbm, out_hbm, didx2, ones_v, acc):
        c = lax.axis_index("c")
        s = lax.axis_index("s")
        _slab_init(z_hbm, acc, s, 0)
        pltpu.sync_copy(one_hbm, ones_v)
        plsc.subcore_barrier()

        base_row = (c * NS + s) * EPT2_CHUNKS

        def body(g, carry):
            pltpu.sync_copy(dst2_hbm.at[pl.ds(base_row + g * GRP, GRP)], didx2)
            for j in range(GRP):
                pltpu.sync_copy(ones_v, acc.at[didx2.at[j]], add=True)
            return carry

        lax.fori_loop(0, EPT2_CHUNKS // GRP, body, 0)
        plsc.subcore_barrier()
        _slab_out(acc, out_hbm, s, c * N)

    return k(dst_pad, zeros128, ones128)


def _sc_agg(y_cat, src_flat, dst_pad):
    D = 128

    @functools.partial(
        pl.kernel,
        out_type=jax.ShapeDtypeStruct((2 * N, D), jnp.float32),
        mesh=_mesh(),
        scratch_types=_agg_scratch(D),
    )
    def k(y_hbm, src_hbm, dst_hbm, out_hbm, *sc):
        sidx2, didx2, rows = sc[0], sc[1], sc[2:4]
        acc = sc[4]
        semg = sc[5:7]
        c = lax.axis_index("c")
        s = lax.axis_index("s")
        _slab_init(y_hbm, acc, s, c * N)
        plsc.subcore_barrier()
        _agg_pipeline(
            y_hbm, src_hbm, dst_hbm, acc, sidx2, didx2, rows, semg,
            c * (E_PAD // C) + s * EPT_CHUNKS, s * EPT_CHUNKS, EPT_CHUNKS,
        )
        plsc.subcore_barrier()
        _slab_out(acc, out_hbm, s, c * N)

    return k(y_cat, src_flat, dst_pad)


def _sc_agg_edges(y, src_pad, dst_pad, zeros128):
    D = 128

    @functools.partial(
        pl.kernel,
        out_type=jax.ShapeDtypeStruct((2 * N, D), jnp.float32),
        mesh=_mesh(),
        scratch_types=_agg_scratch(D),
    )
    def k(y_hbm, z_hbm, src_hbm, dst_hbm, out_hbm, *sc):
        sidx2, didx2, rows = sc[0], sc[1], sc[2:4]
        acc = sc[4]
        semg = sc[5:7]
        c = lax.axis_index("c")
        s = lax.axis_index("s")

        @pl.when(c == 0)
        def _():
            _slab_init(y_hbm, acc, s, 0)

        @pl.when(c == 1)
        def _():
            _slab_init(z_hbm, acc, s, 0)

        plsc.subcore_barrier()
        base_row = (c * NS + s) * EPT2_CHUNKS
        _agg_pipeline(
            y_hbm, src_hbm, dst_hbm, acc, sidx2, didx2, rows, semg,
            base_row, base_row, EPT2_CHUNKS,
        )
        plsc.subcore_barrier()
        _slab_out(acc, out_hbm, s, c * N)

    return k(y, zeros128, src_pad, dst_pad)


def _tc_scale_mm(x, w, degp):
    K = x.shape[1]
    F = w.shape[1] // 2

    def body(x_ref, w_ref, dlo_ref, dhi_ref, o_ref):
        dis = lax.rsqrt(dlo_ref[:, 0:1] + dhi_ref[:, 0:1] + 1.0)
        o_ref[...] = dis * jnp.dot(
            x_ref[...], w_ref[...], preferred_element_type=jnp.float32
        )

    return pl.pallas_call(
        body,
        grid=(NB, 2),
        in_specs=[
            pl.BlockSpec((R, K), lambda i, c: (i, 0)),
            pl.BlockSpec((K, F), lambda i, c: (0, c)),
            pl.BlockSpec((R, 128), lambda i, c: (i, 0)),
            pl.BlockSpec((R, 128), lambda i, c: (NB + i, 0)),
        ],
        out_specs=pl.BlockSpec((R, F), lambda i, c: (c * NB + i, 0)),
        out_shape=jax.ShapeDtypeStruct((2 * N, F), jnp.float32),
    )(x, w, degp, degp)


def _tc_mid(agg, degp, b1, w2):
    def body(lo_ref, hi_ref, dlo_ref, dhi_ref, b_ref, w_ref, o_ref):
        dis = lax.rsqrt(dlo_ref[:, 0:1] + dhi_ref[:, 0:1] + 1.0)
        h = jnp.concatenate([lo_ref[...], hi_ref[...]], axis=1)
        h = jnp.maximum(dis * h + b_ref[...], 0.0)
        o_ref[...] = dis * jnp.dot(h, w_ref[...], preferred_element_type=jnp.float32)

    return pl.pallas_call(
        body,
        grid=(NB,),
        in_specs=[
            pl.BlockSpec((R, 128), lambda i: (i, 0)),
            pl.BlockSpec((R, 128), lambda i: (NB + i, 0)),
            pl.BlockSpec((R, 128), lambda i: (i, 0)),
            pl.BlockSpec((R, 128), lambda i: (NB + i, 0)),
            pl.BlockSpec((1, 256), lambda i: (0, 0)),
            pl.BlockSpec((256, 128), lambda i: (0, 0)),
        ],
        out_specs=pl.BlockSpec((R, 128), lambda i: (i, 0)),
        out_shape=jax.ShapeDtypeStruct((N, 128), jnp.float32),
    )(agg, agg, degp, degp, b1, w2)


def _tc_final(agg, degp, b2):
    def body(lo_ref, hi_ref, dlo_ref, dhi_ref, b_ref, o_ref):
        dis = lax.rsqrt(dlo_ref[:, 0:1] + dhi_ref[:, 0:1] + 1.0)
        h = lo_ref[...] + hi_ref[...]
        o_ref[...] = jnp.maximum(dis * h + b_ref[...], 0.0)

    return pl.pallas_call(
        body,
        grid=(NB,),
        in_specs=[
            pl.BlockSpec((R, 128), lambda i: (i, 0)),
            pl.BlockSpec((R, 128), lambda i: (NB + i, 0)),
            pl.BlockSpec((R, 128), lambda i: (i, 0)),
            pl.BlockSpec((R, 128), lambda i: (NB + i, 0)),
            pl.BlockSpec((1, 128), lambda i: (0, 0)),
        ],
        out_specs=pl.BlockSpec((R, 128), lambda i: (i, 0)),
        out_shape=jax.ShapeDtypeStruct((N, 128), jnp.float32),
    )(agg, agg, degp, degp, b2)


def kernel(x, edge_index, W1, b1, W2, b2):
    src = edge_index[0].astype(jnp.int32)
    dst = edge_index[1].astype(jnp.int32)
    pad = E_PAD - E
    src_p = jnp.concatenate([src, jnp.zeros((pad,), jnp.int32)])
    dst_p = jnp.concatenate([dst, jnp.full((pad,), N, jnp.int32)])
    src_flat = jnp.concatenate([src_p, src_p + N])
    zeros128 = jnp.zeros((N, 128), jnp.float32)
    ones128 = jnp.ones((C, 128), jnp.float32)

    src2_flat = src_flat.reshape(2 * E_PAD // C, C)
    src2 = src_p.reshape(E_PAD // C, C)
    dst2 = dst_p.reshape(E_PAD // C, C)

    degp = _sc_deg(dst2, zeros128, ones128)
    y1 = _tc_scale_mm(x, W1, degp)
    a1 = _sc_agg(y1, src2_flat, dst2)
    y2 = _tc_mid(a1, degp, b1.reshape(1, -1), W2)
    a2 = _sc_agg_edges(y2, src2, dst2, zeros128)
    return _tc_final(a2, degp, b2.reshape(1, -1))
